# baseline (speedup 1.0000x reference)
import dataclasses
import functools

import jax
import jax.numpy as jnp
from jax import lax
from jax.experimental import pallas as pl
from jax.experimental.pallas import tpu as pltpu
from jax.experimental.pallas import tpu_sc as plsc

NUM_CORES = 2
NUM_SUBCORES = 16
NW = NUM_CORES * NUM_SUBCORES
LANES = 16

BATCH = 16384
DIM = 64
B_PER_W = BATCH // NW


def _mf_kernel(ids_hbm, vals_hbm, table_hbm, out_hbm,
               ids2_v, vals2_v, idx_v, rows_v, v01_v, out_v, part_v, sem):
    wid = lax.axis_index("s") * NUM_CORES + lax.axis_index("c")
    base = wid * B_PER_W

    pltpu.sync_copy(ids_hbm.at[pl.ds(base, B_PER_W), :], ids2_v)
    pltpu.sync_copy(vals_hbm.at[pl.ds(base, B_PER_W), :], vals2_v)

    lane_iota = lax.iota(jnp.int32, LANES)
    zeros16 = jnp.full((LANES,), 0, jnp.int32)
    ones16 = jnp.full((LANES,), 1, jnp.int32)

    @pl.loop(0, 2 * B_PER_W, step=LANES)
    def _(k):
        kv = k + lane_iota
        idx_v[pl.ds(k, LANES)] = plsc.load_gather(
            ids2_v, [lax.shift_right_logical(kv, 1), lax.bitwise_and(kv, 1)])

    gather = pltpu.async_copy(table_hbm.at[idx_v], rows_v, sem)


    @pl.loop(0, B_PER_W, step=LANES)
    def _(g):
        rows16 = g + lane_iota
        ve = plsc.load_gather(vals2_v, [rows16, zeros16])
        vo = plsc.load_gather(vals2_v, [rows16, ones16])
        v01_v[pl.ds(g, LANES)] = ve * vo

    gather.wait()

    @pl.loop(0, B_PER_W, step=LANES)
    def _(g):
        for r in range(LANES):
            b2 = 2 * (g + r)
            part = rows_v[b2, pl.ds(0, LANES)] * rows_v[b2 + 1, pl.ds(0, LANES)]
            for d in range(LANES, DIM, LANES):
                part += rows_v[b2, pl.ds(d, LANES)] * rows_v[b2 + 1, pl.ds(d, LANES)]
            part_v[r, pl.ds(0, LANES)] = part
        acc = plsc.load_gather(part_v, [lane_iota, zeros16])
        for c in range(1, LANES):
            acc += plsc.load_gather(part_v, [lane_iota, jnp.full((LANES,), c, jnp.int32)])
        sl = pl.ds(g, LANES)
        out_v[sl] = acc * v01_v[sl]

    pltpu.sync_copy(out_v, out_hbm.at[pl.ds(base, B_PER_W)])


@jax.jit
def kernel(feature_ids, feature_vals, table):
    mesh = plsc.VectorSubcoreMesh(core_axis_name="c", subcore_axis_name="s")
    cp = pltpu.CompilerParams()
    for fld, val in (("needs_layout_passes", False),
                     ("use_tc_tiling_on_sc", False)):
        if fld in pltpu.CompilerParams.__dataclass_fields__:
            cp = dataclasses.replace(cp, **{fld: val})
    run = functools.partial(
        pl.kernel,
        mesh=mesh,
        compiler_params=cp,
        out_type=jax.ShapeDtypeStruct((BATCH,), jnp.float32),
        scratch_types=[
            pltpu.VMEM((B_PER_W, 2), jnp.int32),
            pltpu.VMEM((B_PER_W, 2), jnp.float32),
            pltpu.VMEM((2 * B_PER_W,), jnp.int32),
            pltpu.VMEM((2 * B_PER_W, DIM), jnp.float32),
            pltpu.VMEM((B_PER_W,), jnp.float32),
            pltpu.VMEM((B_PER_W,), jnp.float32),
            pltpu.VMEM((LANES, LANES), jnp.float32),
            pltpu.SemaphoreType.DMA,
        ],
    )(_mf_kernel)
    return run(feature_ids, feature_vals, table)


# speedup vs baseline: 1.2629x; 1.2629x over previous
import dataclasses
import functools

import jax
import jax.numpy as jnp
from jax import lax
from jax.experimental import pallas as pl
from jax.experimental.pallas import tpu as pltpu
from jax.experimental.pallas import tpu_sc as plsc

NUM_CORES = 2
NUM_SUBCORES = 16
NW = NUM_CORES * NUM_SUBCORES
LANES = 16

BATCH = 16384
DIM = 64
B_PER_W = BATCH // NW
I_PER_W = 2 * B_PER_W


def _make_params(tc_tiling):
    cp = pltpu.CompilerParams()
    for fld, val in (("needs_layout_passes", False),
                     ("use_tc_tiling_on_sc", tc_tiling)):
        if fld in pltpu.CompilerParams.__dataclass_fields__:
            cp = dataclasses.replace(cp, **{fld: val})
    return cp


PREP_CHUNK = 128
PREP_STEPS = B_PER_W // PREP_CHUNK


def _prep_kernel(ids_hbm, vals_hbm, idx_hbm, v01_hbm,
                 ids2_v, vals2_v, idx_v, v01_v):
    wid = lax.axis_index("s") * NUM_CORES + lax.axis_index("c")
    base = wid * B_PER_W

    lane_iota = lax.iota(jnp.int32, LANES)
    zeros16 = jnp.full((LANES,), 0, jnp.int32)
    ones16 = jnp.full((LANES,), 1, jnp.int32)

    for c in range(PREP_STEPS):
        r0 = base + c * PREP_CHUNK
        pltpu.sync_copy(ids_hbm.at[pl.ds(r0, PREP_CHUNK), :], ids2_v)
        pltpu.sync_copy(vals_hbm.at[pl.ds(r0, PREP_CHUNK), :], vals2_v)

        @pl.loop(0, 2 * PREP_CHUNK, step=LANES)
        def _(k):
            kv = k + lane_iota
            idx_v[pl.ds(2 * PREP_CHUNK * c + k, LANES)] = plsc.load_gather(
                ids2_v, [lax.shift_right_logical(kv, 1), lax.bitwise_and(kv, 1)])

        @pl.loop(0, PREP_CHUNK, step=LANES)
        def _(g):
            rows16 = g + lane_iota
            ve = plsc.load_gather(vals2_v, [rows16, zeros16])
            vo = plsc.load_gather(vals2_v, [rows16, ones16])
            v01_v[pl.ds(PREP_CHUNK * c + g, LANES)] = ve * vo

    pltpu.sync_copy(idx_v, idx_hbm.at[pl.ds(2 * base, I_PER_W)])
    pltpu.sync_copy(v01_v, v01_hbm.at[pl.ds(base, B_PER_W)])


def _mf_kernel(idx_hbm, v01_hbm, table_hbm, out_hbm,
               idx_v, rows_v, v01_v, out_v, part_v, sem):
    wid = lax.axis_index("s") * NUM_CORES + lax.axis_index("c")
    base = wid * B_PER_W

    pltpu.sync_copy(idx_hbm.at[pl.ds(2 * base, I_PER_W)], idx_v)
    gather = pltpu.async_copy(table_hbm.at[idx_v], rows_v, sem)
    pltpu.sync_copy(v01_hbm.at[pl.ds(base, B_PER_W)], v01_v)

    lane_iota = lax.iota(jnp.int32, LANES)
    zeros16 = jnp.full((LANES,), 0, jnp.int32)
    gather.wait()

    @pl.loop(0, B_PER_W, step=LANES)
    def _(g):
        for r in range(LANES):
            b2 = 2 * (g + r)
            part = rows_v[b2, pl.ds(0, LANES)] * rows_v[b2 + 1, pl.ds(0, LANES)]
            for d in range(LANES, DIM, LANES):
                part += rows_v[b2, pl.ds(d, LANES)] * rows_v[b2 + 1, pl.ds(d, LANES)]
            part_v[r, pl.ds(0, LANES)] = part
        acc = plsc.load_gather(part_v, [lane_iota, zeros16])
        for c in range(1, LANES):
            acc += plsc.load_gather(part_v, [lane_iota, jnp.full((LANES,), c, jnp.int32)])
        sl = pl.ds(g, LANES)
        out_v[sl] = acc * v01_v[sl]

    pltpu.sync_copy(out_v, out_hbm.at[pl.ds(base, B_PER_W)])


@jax.jit
def kernel(feature_ids, feature_vals, table):
    mesh = plsc.VectorSubcoreMesh(core_axis_name="c", subcore_axis_name="s")

    prep = functools.partial(
        pl.kernel,
        mesh=mesh,
        compiler_params=_make_params(True),
        out_type=(
            jax.ShapeDtypeStruct((2 * BATCH,), jnp.int32),
            jax.ShapeDtypeStruct((BATCH,), jnp.float32),
        ),
        scratch_types=[
            pltpu.VMEM((PREP_CHUNK, 2), jnp.int32),
            pltpu.VMEM((PREP_CHUNK, 2), jnp.float32),
            pltpu.VMEM((I_PER_W,), jnp.int32),
            pltpu.VMEM((B_PER_W,), jnp.float32),
        ],
    )(_prep_kernel)
    idx_flat, v01 = prep(feature_ids, feature_vals)

    run = functools.partial(
        pl.kernel,
        mesh=mesh,
        compiler_params=_make_params(False),
        out_type=jax.ShapeDtypeStruct((BATCH,), jnp.float32),
        scratch_types=[
            pltpu.VMEM((I_PER_W,), jnp.int32),
            pltpu.VMEM((I_PER_W, DIM), jnp.float32),
            pltpu.VMEM((B_PER_W,), jnp.float32),
            pltpu.VMEM((B_PER_W,), jnp.float32),
            pltpu.VMEM((LANES, LANES), jnp.float32),
            pltpu.SemaphoreType.DMA,
        ],
    )(_mf_kernel)
    return run(idx_flat, v01, table)
